# baseline (speedup 1.0000x reference)
import functools
import math

import jax
import jax.numpy as jnp
from jax import lax
from jax.experimental import pallas as pl
from jax.experimental.pallas import tpu as pltpu
from jax.experimental.pallas import tpu_sc as plsc

_N_FIELDS = 26
_VOCAB = 100000
_EMB_DIM = 32
_D_MODEL = 1024
_BATCH = 4096
_SUM_EMB = _N_FIELDS * _EMB_DIM
_SCALE = math.sqrt(_EMB_DIM)

_NC = 2
_NS = 16
_NW = _NC * _NS
_L = 16

_ROWS = _BATCH * _N_FIELDS
_RPW = _ROWS // _NW
_CHUNK = 128
_NCHUNK = _RPW // _CHUNK


def _gather_body(x_hbm, tab_hbm, out_hbm, xv, idxv, rows_v, sem):
    wid = lax.axis_index("s") * _NC + lax.axis_index("c")
    base = wid * _RPW
    pltpu.sync_copy(x_hbm.at[pl.ds(base, _RPW)], xv)

    lanes = lax.iota(jnp.int32, _L)

    def fld_body(r, carry):
        for v in range(_CHUNK // _L):
            off = r * _CHUNK + v * _L
            pos = off + lanes
            fld = lax.rem(pos, _N_FIELDS)
            idxv[r, pl.ds(v * _L, _L)] = xv[pl.ds(off, _L)] + fld * _VOCAB
        return carry

    lax.fori_loop(0, _NCHUNK, fld_body, 0)

    copies = []
    for r in range(_NCHUNK):
        c = pltpu.make_async_copy(
            tab_hbm.at[idxv.at[r]],
            rows_v.at[pl.ds(r * _CHUNK, _CHUNK)],
            sem,
        )
        c.start()
        copies.append(c)
    for c in copies:
        c.wait()

    pltpu.sync_copy(rows_v, out_hbm.at[pl.ds(base, _RPW)])


@functools.cache
def _make_gather():
    return pl.kernel(
        _gather_body,
        out_type=jax.ShapeDtypeStruct((_ROWS, _EMB_DIM), jnp.float32),
        mesh=plsc.VectorSubcoreMesh(core_axis_name="c", subcore_axis_name="s"),
        scratch_types=[
            pltpu.VMEM((_RPW,), jnp.int32),
            pltpu.VMEM((_NCHUNK, _CHUNK), jnp.int32),
            pltpu.VMEM((_RPW, _EMB_DIM), jnp.float32),
            pltpu.SemaphoreType.DMA,
        ],
        compiler_params=pltpu.CompilerParams(use_tc_tiling_on_sc=False),
    )


def _proj_body(e_ref, w_ref, b_ref, o_ref):
    acc = jnp.dot(e_ref[...], w_ref[...], preferred_element_type=jnp.float32)
    o_ref[...] = acc * _SCALE + b_ref[...]


_M_TILE = 512

_proj = pl.pallas_call(
    _proj_body,
    grid=(_BATCH // _M_TILE,),
    in_specs=[
        pl.BlockSpec((_M_TILE, _SUM_EMB), lambda i: (i, 0)),
        pl.BlockSpec((_SUM_EMB, _D_MODEL), lambda i: (0, 0)),
        pl.BlockSpec((1, _D_MODEL), lambda i: (0, 0)),
    ],
    out_specs=pl.BlockSpec((_M_TILE, _D_MODEL), lambda i: (i, 0)),
    out_shape=jax.ShapeDtypeStruct((_BATCH, _D_MODEL), jnp.float32),
)


def kernel(x, tables, W, b):
    x_flat = x.reshape(_ROWS)
    tab_flat = tables.reshape(_N_FIELDS * _VOCAB, _EMB_DIM)
    embs = _make_gather()(x_flat, tab_flat).reshape(_BATCH, _SUM_EMB)
    return _proj(embs, W, b.reshape(1, _D_MODEL))
